# baseline (speedup 1.0000x reference)
import functools

import jax
import jax.numpy as jnp
from jax import lax
from jax.experimental import pallas as pl
from jax.experimental.pallas import tpu as pltpu
from jax.experimental.pallas import tpu_sc as plsc

N = 10000
E = 320000
D = 128
G = 256
L = 3

NP = 10240
DH = D // 2
CH = 80
NCH = 252
EPS = NCH * CH
EP = 16 * EPS
RPS = NP // 16

BR = 1280
NB = NP // BR
GB = NP // G

_f32 = jnp.float32


def _sc_scatter_body(mA_hbm, mB_hbm, src_hbm, dst_hbm, z_hbm, out_hbm,
                     src_v, dst_v, rows0, rows1, rows2, rows3, rows4, rows5,
                     acc_sh, gsem0, gsem1, gsem2, gsem3, gsem4, gsem5,
                     ssem0, ssem1, ssem2, ssem3, ssem4, ssem5):
    c = lax.axis_index("c")
    s = lax.axis_index("s")
    bufs = (rows0, rows1, rows2, rows3, rows4, rows5)
    gsems = (gsem0, gsem1, gsem2, gsem3, gsem4, gsem5)
    ssems = (ssem0, ssem1, ssem2, ssem3, ssem4, ssem5)

    pltpu.sync_copy(z_hbm, acc_sh.at[pl.ds(s * RPS, RPS)])
    pltpu.sync_copy(src_hbm.at[s], src_v)
    pltpu.sync_copy(dst_hbm.at[s], dst_v)
    plsc.subcore_barrier()

    def run(m_hbm):
        def start_g(j, t):
            pltpu.async_copy(m_hbm.at[src_v.at[j]], bufs[t], gsems[t])

        def wait_g(j, t):
            pltpu.make_async_copy(m_hbm.at[src_v.at[j]], bufs[t],
                                  gsems[t]).wait()

        def start_s(j, t):
            pltpu.async_copy(bufs[t], acc_sh.at[dst_v.at[j]], ssems[t],
                             add=True)

        def wait_s(j, t):
            pltpu.make_async_copy(bufs[t], acc_sh.at[dst_v.at[j]],
                                  ssems[t]).wait()

        start_g(0, 0)
        start_g(1, 1)

        @pl.loop(0, NCH, step=6)
        def _(j):
            for k in range(6):
                wait_g(j + k, k)
                start_s(j + k, k)
                t2 = (k + 2) % 6

                @pl.when(j + k + 2 < NCH)
                def _(j=j, k=k, t2=t2):
                    if k < 4:
                        @pl.when(j > 0)
                        def _():
                            wait_s(j + k - 4, t2)
                    else:
                        wait_s(j + k - 4, t2)
                    start_g(j + k + 2, t2)

        for k in range(6):
            wait_s(NCH - 6 + k, k)

    @pl.when(c == 0)
    def _():
        run(mA_hbm)

    @pl.when(c == 1)
    def _():
        run(mB_hbm)

    plsc.subcore_barrier()
    pltpu.sync_copy(acc_sh.at[pl.ds(s * RPS, RPS)],
                    out_hbm.at[pl.ds(c * NP + s * RPS, RPS)])


def _sc_scatter_agg(mA, mB, src, dst, zeros):
    mesh = plsc.VectorSubcoreMesh(core_axis_name="c", subcore_axis_name="s")
    return pl.kernel(
        _sc_scatter_body,
        out_type=jax.ShapeDtypeStruct((2 * NP, DH), _f32),
        mesh=mesh,
        compiler_params=pltpu.CompilerParams(use_tc_tiling_on_sc=False),
        scratch_types=(
            [pltpu.VMEM((NCH, CH), jnp.int32)] * 2
            + [pltpu.VMEM((CH, DH), _f32)] * 6
            + [pltpu.VMEM_SHARED((NP, DH), _f32)]
            + [pltpu.SemaphoreType.DMA] * 12
        ),
    )(mA, mB, src, dst, zeros)


def _dot(a, b):
    return jnp.dot(a, b, preferred_element_type=_f32)


def _tc_input_body(x_ref, winT_ref, bin_ref, cw0_ref, h_ref, mA_ref, mB_ref):
    h = _dot(x_ref[...], winT_ref[...]) + bin_ref[...]
    h_ref[...] = h
    m = _dot(h, cw0_ref[...])
    mA_ref[...] = m[:, :DH]
    mB_ref[...] = m[:, DH:]


def _tc_input(xp, winT, binr, cw0):
    row = pl.BlockSpec((BR, D), lambda i: (i, 0))
    rowh = pl.BlockSpec((BR, DH), lambda i: (i, 0))
    full = lambda shape: pl.BlockSpec(shape, lambda i: (0, 0))
    return pl.pallas_call(
        _tc_input_body,
        grid=(NB,),
        in_specs=[row, full((D, D)), full((1, D)), full((D, D))],
        out_specs=[row, rowh, rowh],
        out_shape=[jax.ShapeDtypeStruct((NP, D), _f32),
                   jax.ShapeDtypeStruct((NP, DH), _f32),
                   jax.ShapeDtypeStruct((NP, DH), _f32)],
    )(xp, winT, binr, cw0)


def _tc_gh_body(h_ref, whhT_ref, bhh_ref, gh_ref):
    gh_ref[...] = _dot(h_ref[...], whhT_ref[...]) + bhh_ref[...]


def _tc_gh(h, whhT, bhh):
    return pl.pallas_call(
        _tc_gh_body,
        grid=(NB,),
        in_specs=[pl.BlockSpec((BR, D), lambda i: (i, 0)),
                  pl.BlockSpec((D, 3 * D), lambda i: (0, 0)),
                  pl.BlockSpec((1, 3 * D), lambda i: (0, 0))],
        out_specs=pl.BlockSpec((BR, 3 * D), lambda i: (i, 0)),
        out_shape=jax.ShapeDtypeStruct((NP, 3 * D), _f32),
    )(h, whhT, bhh)


def _gru_update(agg, h, gh, wihT, bih):
    gi = _dot(agg, wihT) + bih
    r = jax.nn.sigmoid(gi[:, :D] + gh[:, :D])
    z = jax.nn.sigmoid(gi[:, D:2 * D] + gh[:, D:2 * D])
    n = jnp.tanh(gi[:, 2 * D:] + r * gh[:, 2 * D:])
    return (1.0 - z) * n + z * h


def _tc_gru_body(pA_ref, pB_ref, h_ref, gh_ref, wihT_ref, bih_ref, cwn_ref,
                 hn_ref, mAn_ref, mBn_ref):
    agg = jnp.concatenate([pA_ref[...], pB_ref[...]], axis=1)
    hn = _gru_update(agg, h_ref[...], gh_ref[...],
                     wihT_ref[...], bih_ref[...])
    hn_ref[...] = hn
    mn = _dot(hn, cwn_ref[...])
    mAn_ref[...] = mn[:, :DH]
    mBn_ref[...] = mn[:, DH:]


def _tc_gru(parts, h, gh, wihT, bih, cwn):
    row = pl.BlockSpec((BR, D), lambda i: (i, 0))
    rowh = pl.BlockSpec((BR, DH), lambda i: (i, 0))
    full = lambda shape: pl.BlockSpec(shape, lambda i: (0, 0))
    return pl.pallas_call(
        _tc_gru_body,
        grid=(NB,),
        in_specs=[pl.BlockSpec((BR, DH), lambda i: (i, 0)),
                  pl.BlockSpec((BR, DH), lambda i: (i + NB, 0)),
                  row,
                  pl.BlockSpec((BR, 3 * D), lambda i: (i, 0)),
                  full((D, 3 * D)), full((1, 3 * D)), full((D, D))],
        out_specs=[row, rowh, rowh],
        out_shape=[jax.ShapeDtypeStruct((NP, D), _f32),
                   jax.ShapeDtypeStruct((NP, DH), _f32),
                   jax.ShapeDtypeStruct((NP, DH), _f32)],
    )(parts, parts, h, gh, wihT, bih, cwn)


def _tc_gru_last_body(pA_ref, pB_ref, h_ref, gh_ref, wihT_ref, bih_ref,
                      hn_ref):
    agg = jnp.concatenate([pA_ref[...], pB_ref[...]], axis=1)
    hn_ref[...] = _gru_update(agg, h_ref[...],
                              gh_ref[...], wihT_ref[...], bih_ref[...])


def _tc_gru_last(parts, h, gh, wihT, bih):
    row = pl.BlockSpec((BR, D), lambda i: (i, 0))
    full = lambda shape: pl.BlockSpec(shape, lambda i: (0, 0))
    return pl.pallas_call(
        _tc_gru_last_body,
        grid=(NB,),
        in_specs=[pl.BlockSpec((BR, DH), lambda i: (i, 0)),
                  pl.BlockSpec((BR, DH), lambda i: (i + NB, 0)),
                  row,
                  pl.BlockSpec((BR, 3 * D), lambda i: (i, 0)),
                  full((D, 3 * D)), full((1, 3 * D))],
        out_specs=row,
        out_shape=jax.ShapeDtypeStruct((NP, D), _f32),
    )(parts, parts, h, gh, wihT, bih)


def _tc_pool_body(h_ref, b_ref, gateT_ref, gateb_ref, fc1T_ref, fc1b_ref,
                  fc2T_ref, fc2b_ref, outT_ref, outb_ref, o_ref):
    iota_g = lax.broadcasted_iota(jnp.int32, (1, G), 1)
    gateT = gateT_ref[...]
    gate_b = gateb_ref[...]

    def block(b):
        rows = pl.ds(b * G, G)
        hb = h_ref[rows, :]
        bcol = b_ref[rows, :]
        mask = bcol == iota_g
        gb = _dot(hb, gateT) + gate_b
        return hb, mask, gb

    def pass1(b, gmax):
        _, mask, gb = block(b)
        m = jnp.max(jnp.where(mask, gb, -1e30), axis=0, keepdims=True)
        return jnp.maximum(gmax, m)

    gmax = lax.fori_loop(0, GB, pass1, jnp.full((1, G), -1e30, _f32))

    def pass2(b, carry):
        den, pooled = carry
        hb, mask, gb = block(b)
        maskf = mask.astype(_f32)
        gmaxn = lax.dot_general(maskf, gmax, (((1,), (1,)), ((), ())),
                                preferred_element_type=_f32)
        w = jnp.exp(gb - gmaxn)
        den = den + lax.dot_general(maskf, w, (((0,), (0,)), ((), ())),
                                    preferred_element_type=_f32)
        pooled = pooled + lax.dot_general(maskf * w, hb,
                                          (((0,), (0,)), ((), ())),
                                          preferred_element_type=_f32)
        return den, pooled

    den, pooled = lax.fori_loop(
        0, GB, pass2,
        (jnp.zeros((G, 1), _f32), jnp.zeros((G, D), _f32)))
    pooled = pooled / (den + 1e-16)
    o = jax.nn.relu(_dot(pooled, fc1T_ref[...]) + fc1b_ref[...])
    o = jax.nn.relu(_dot(o, fc2T_ref[...]) + fc2b_ref[...])
    o_ref[...] = _dot(o, outT_ref[...]) + outb_ref[...]


def _tc_pool(h, batch_col, gateT, gate_b, fc1T, fc1b, fc2T, fc2b, outT, outb):
    vm = lambda: pl.BlockSpec(memory_space=pltpu.VMEM)
    return pl.pallas_call(
        _tc_pool_body,
        in_specs=[vm() for _ in range(10)],
        out_specs=vm(),
        out_shape=jax.ShapeDtypeStruct((G, 1), _f32),
    )(h, batch_col, gateT, gate_b, fc1T, fc1b, fc2T, fc2b, outT, outb)


def kernel(x, edge_index, batch, W_in, b_in, conv_w, gru_Wih, gru_Whh,
           gru_bih, gru_bhh, gate_W, gate_b, fc1_W, fc1_b, fc2_W, fc2_b,
           out_W, out_b):
    xp = jnp.pad(x, ((0, NP - N), (0, 0)))
    src = jnp.pad(edge_index[0], (0, EP - E)).reshape(16, NCH, CH)
    dump = (N + jnp.arange(EP - E, dtype=jnp.int32) % (NP - N)).astype(jnp.int32)
    dst = jnp.concatenate([edge_index[1], dump]).reshape(16, NCH, CH)
    batch_col = jnp.pad(batch, (0, NP - N), constant_values=G).reshape(NP, 1)
    zeros = jnp.zeros((RPS, DH), _f32)

    winT = W_in.T
    wihT = gru_Wih.T
    whhT = gru_Whh.T
    binr = b_in.reshape(1, D)
    bih = gru_bih.reshape(1, 3 * D)
    bhh = gru_bhh.reshape(1, 3 * D)

    h, mA, mB = _tc_input(xp, winT, binr, conv_w[0])
    for i in range(L):
        parts = _sc_scatter_agg(mA, mB, src, dst, zeros)
        gh = _tc_gh(h, whhT, bhh)
        if i < L - 1:
            h, mA, mB = _tc_gru(parts, h, gh, wihT, bih, conv_w[i + 1])
        else:
            h = _tc_gru_last(parts, h, gh, wihT, bih)

    out = _tc_pool(h, batch_col, gate_W.T, gate_b.reshape(1, 1),
                   fc1_W.T, fc1_b.reshape(1, D), fc2_W.T, fc2_b.reshape(1, D),
                   out_W.T, out_b.reshape(1, 1))
    return out.reshape(-1)


# speedup vs baseline: 1.2091x; 1.2091x over previous
import functools

import jax
import jax.numpy as jnp
from jax import lax
from jax.experimental import pallas as pl
from jax.experimental.pallas import tpu as pltpu
from jax.experimental.pallas import tpu_sc as plsc

N = 10000
E = 320000
D = 128
G = 256
L = 3

NP = 10240
DH = D // 2
CH = 80
NCH = 250
EPS = NCH * CH
EP = 16 * EPS
RPS = NP // 16

BR = 1280
NB = NP // BR
PB = 1024
GB = NP // PB

_f32 = jnp.float32


def _sc_scatter_body(mA_hbm, mB_hbm, src_hbm, dst_hbm, z_hbm, out_hbm,
                     src_v, dst_v, rows0, rows1, acc_sh, gsem0, gsem1):
    c = lax.axis_index("c")
    s = lax.axis_index("s")
    bufs = (rows0, rows1)
    gsems = (gsem0, gsem1)

    pltpu.sync_copy(z_hbm, acc_sh.at[pl.ds(s * RPS, RPS)])
    pltpu.sync_copy(src_hbm.at[s], src_v)
    pltpu.sync_copy(dst_hbm.at[s], dst_v)
    plsc.subcore_barrier()

    def run(m_hbm):
        def start_g(j, t):
            pltpu.async_copy(m_hbm.at[src_v.at[j]], bufs[t], gsems[t])

        def wait_g(j, t):
            pltpu.make_async_copy(m_hbm.at[src_v.at[j]], bufs[t],
                                  gsems[t]).wait()

        def scat(j, t):
            pltpu.sync_copy(bufs[t], acc_sh.at[dst_v.at[j]], add=True)

        start_g(0, 0)

        @pl.loop(0, NCH, step=2)
        def _(j):
            start_g(j + 1, 1)
            wait_g(j, 0)
            scat(j, 0)

            @pl.when(j + 2 < NCH)
            def _():
                start_g(j + 2, 0)

            wait_g(j + 1, 1)
            scat(j + 1, 1)

    @pl.when(c == 0)
    def _():
        run(mA_hbm)

    @pl.when(c == 1)
    def _():
        run(mB_hbm)

    plsc.subcore_barrier()
    pltpu.sync_copy(acc_sh.at[pl.ds(s * RPS, RPS)],
                    out_hbm.at[pl.ds(c * NP + s * RPS, RPS)])


def _sc_scatter_agg(mA, mB, src, dst, zeros):
    mesh = plsc.VectorSubcoreMesh(core_axis_name="c", subcore_axis_name="s")
    return pl.kernel(
        _sc_scatter_body,
        out_type=jax.ShapeDtypeStruct((2 * NP, DH), _f32),
        mesh=mesh,
        compiler_params=pltpu.CompilerParams(use_tc_tiling_on_sc=False),
        scratch_types=(
            [pltpu.VMEM((NCH, CH), jnp.int32)] * 2
            + [pltpu.VMEM((CH, DH), _f32)] * 2
            + [pltpu.VMEM_SHARED((NP, DH), _f32)]
            + [pltpu.SemaphoreType.DMA] * 2
        ),
    )(mA, mB, src, dst, zeros)


def _dot(a, b):
    return jnp.dot(a, b, preferred_element_type=_f32)


def _tc_input_body(x_ref, winT_ref, bin_ref, cw0_ref, h_ref, mA_ref, mB_ref):
    h = _dot(x_ref[...], winT_ref[...]) + bin_ref[...]
    h_ref[...] = h
    m = _dot(h, cw0_ref[...])
    mA_ref[...] = m[:, :DH]
    mB_ref[...] = m[:, DH:]


def _tc_input(xp, winT, binr, cw0):
    row = pl.BlockSpec((BR, D), lambda i: (i, 0))
    rowh = pl.BlockSpec((BR, DH), lambda i: (i, 0))
    full = lambda shape: pl.BlockSpec(shape, lambda i: (0, 0))
    return pl.pallas_call(
        _tc_input_body,
        grid=(NB,),
        in_specs=[row, full((D, D)), full((1, D)), full((D, D))],
        out_specs=[row, rowh, rowh],
        out_shape=[jax.ShapeDtypeStruct((NP, D), _f32),
                   jax.ShapeDtypeStruct((NP, DH), _f32),
                   jax.ShapeDtypeStruct((NP, DH), _f32)],
    )(xp, winT, binr, cw0)


def _tc_gh_body(h_ref, whhT_ref, bhh_ref, gh_ref):
    gh_ref[...] = _dot(h_ref[...], whhT_ref[...]) + bhh_ref[...]


def _tc_gh(h, whhT, bhh):
    return pl.pallas_call(
        _tc_gh_body,
        grid=(NB,),
        in_specs=[pl.BlockSpec((BR, D), lambda i: (i, 0)),
                  pl.BlockSpec((D, 3 * D), lambda i: (0, 0)),
                  pl.BlockSpec((1, 3 * D), lambda i: (0, 0))],
        out_specs=pl.BlockSpec((BR, 3 * D), lambda i: (i, 0)),
        out_shape=jax.ShapeDtypeStruct((NP, 3 * D), _f32),
    )(h, whhT, bhh)


def _gru_update(agg, h, gh, wihT, bih):
    gi = _dot(agg, wihT) + bih
    r = jax.nn.sigmoid(gi[:, :D] + gh[:, :D])
    z = jax.nn.sigmoid(gi[:, D:2 * D] + gh[:, D:2 * D])
    n = jnp.tanh(gi[:, 2 * D:] + r * gh[:, 2 * D:])
    return (1.0 - z) * n + z * h


def _tc_gru_body(pA_ref, pB_ref, h_ref, gh_ref, wihT_ref, bih_ref, cwn_ref,
                 hn_ref, mAn_ref, mBn_ref):
    agg = jnp.concatenate([pA_ref[...], pB_ref[...]], axis=1)
    hn = _gru_update(agg, h_ref[...], gh_ref[...],
                     wihT_ref[...], bih_ref[...])
    hn_ref[...] = hn
    mn = _dot(hn, cwn_ref[...])
    mAn_ref[...] = mn[:, :DH]
    mBn_ref[...] = mn[:, DH:]


def _tc_gru(parts, h, gh, wihT, bih, cwn):
    row = pl.BlockSpec((BR, D), lambda i: (i, 0))
    rowh = pl.BlockSpec((BR, DH), lambda i: (i, 0))
    full = lambda shape: pl.BlockSpec(shape, lambda i: (0, 0))
    return pl.pallas_call(
        _tc_gru_body,
        grid=(NB,),
        in_specs=[pl.BlockSpec((BR, DH), lambda i: (i, 0)),
                  pl.BlockSpec((BR, DH), lambda i: (i + NB, 0)),
                  row,
                  pl.BlockSpec((BR, 3 * D), lambda i: (i, 0)),
                  full((D, 3 * D)), full((1, 3 * D)), full((D, D))],
        out_specs=[row, rowh, rowh],
        out_shape=[jax.ShapeDtypeStruct((NP, D), _f32),
                   jax.ShapeDtypeStruct((NP, DH), _f32),
                   jax.ShapeDtypeStruct((NP, DH), _f32)],
    )(parts, parts, h, gh, wihT, bih, cwn)


def _tc_gru_last_body(pA_ref, pB_ref, h_ref, gh_ref, wihT_ref, bih_ref,
                      hn_ref):
    agg = jnp.concatenate([pA_ref[...], pB_ref[...]], axis=1)
    hn_ref[...] = _gru_update(agg, h_ref[...],
                              gh_ref[...], wihT_ref[...], bih_ref[...])


def _tc_gru_last(parts, h, gh, wihT, bih):
    row = pl.BlockSpec((BR, D), lambda i: (i, 0))
    full = lambda shape: pl.BlockSpec(shape, lambda i: (0, 0))
    return pl.pallas_call(
        _tc_gru_last_body,
        grid=(NB,),
        in_specs=[pl.BlockSpec((BR, DH), lambda i: (i, 0)),
                  pl.BlockSpec((BR, DH), lambda i: (i + NB, 0)),
                  row,
                  pl.BlockSpec((BR, 3 * D), lambda i: (i, 0)),
                  full((D, 3 * D)), full((1, 3 * D))],
        out_specs=row,
        out_shape=jax.ShapeDtypeStruct((NP, D), _f32),
    )(parts, parts, h, gh, wihT, bih)


def _tc_pool_body(h_ref, b_ref, gateT_ref, gateb_ref, fc1T_ref, fc1b_ref,
                  fc2T_ref, fc2b_ref, outT_ref, outb_ref, o_ref):
    iota_g = lax.broadcasted_iota(jnp.int32, (1, G), 1)
    gateT = gateT_ref[...]
    gate_b = gateb_ref[...]

    def block(b):
        rows = pl.ds(b * PB, PB)
        hb = h_ref[rows, :]
        bcol = b_ref[rows, :]
        mask = bcol == iota_g
        gb = _dot(hb, gateT) + gate_b
        return hb, mask, gb

    def pass1(b, gmax):
        _, mask, gb = block(b)
        m = jnp.max(jnp.where(mask, gb, -1e30), axis=0, keepdims=True)
        return jnp.maximum(gmax, m)

    gmax = lax.fori_loop(0, GB, pass1, jnp.full((1, G), -1e30, _f32))

    def pass2(b, carry):
        den, pooled = carry
        hb, mask, gb = block(b)
        maskf = mask.astype(_f32)
        gmaxn = lax.dot_general(maskf, gmax, (((1,), (1,)), ((), ())),
                                preferred_element_type=_f32)
        w = jnp.exp(gb - gmaxn)
        den = den + lax.dot_general(maskf, w, (((0,), (0,)), ((), ())),
                                    preferred_element_type=_f32)
        pooled = pooled + lax.dot_general(maskf * w, hb,
                                          (((0,), (0,)), ((), ())),
                                          preferred_element_type=_f32)
        return den, pooled

    den, pooled = lax.fori_loop(
        0, GB, pass2,
        (jnp.zeros((G, 1), _f32), jnp.zeros((G, D), _f32)))
    pooled = pooled / (den + 1e-16)
    o = jax.nn.relu(_dot(pooled, fc1T_ref[...]) + fc1b_ref[...])
    o = jax.nn.relu(_dot(o, fc2T_ref[...]) + fc2b_ref[...])
    o_ref[...] = _dot(o, outT_ref[...]) + outb_ref[...]


def _tc_pool(h, batch_col, gateT, gate_b, fc1T, fc1b, fc2T, fc2b, outT, outb):
    vm = lambda: pl.BlockSpec(memory_space=pltpu.VMEM)
    return pl.pallas_call(
        _tc_pool_body,
        in_specs=[vm() for _ in range(10)],
        out_specs=vm(),
        out_shape=jax.ShapeDtypeStruct((G, 1), _f32),
    )(h, batch_col, gateT, gate_b, fc1T, fc1b, fc2T, fc2b, outT, outb)


def kernel(x, edge_index, batch, W_in, b_in, conv_w, gru_Wih, gru_Whh,
           gru_bih, gru_bhh, gate_W, gate_b, fc1_W, fc1_b, fc2_W, fc2_b,
           out_W, out_b):
    xp = jnp.pad(x, ((0, NP - N), (0, 0)))
    src = edge_index[0].reshape(16, NCH, CH)
    dst = edge_index[1].reshape(16, NCH, CH)
    batch_col = jnp.pad(batch, (0, NP - N), constant_values=G).reshape(NP, 1)
    zeros = jnp.zeros((RPS, DH), _f32)

    winT = W_in.T
    wihT = gru_Wih.T
    whhT = gru_Whh.T
    binr = b_in.reshape(1, D)
    bih = gru_bih.reshape(1, 3 * D)
    bhh = gru_bhh.reshape(1, 3 * D)

    h, mA, mB = _tc_input(xp, winT, binr, conv_w[0])
    for i in range(L):
        parts = _sc_scatter_agg(mA, mB, src, dst, zeros)
        gh = _tc_gh(h, whhT, bhh)
        if i < L - 1:
            h, mA, mB = _tc_gru(parts, h, gh, wihT, bih, conv_w[i + 1])
        else:
            h = _tc_gru_last(parts, h, gh, wihT, bih)

    out = _tc_pool(h, batch_col, gate_W.T, gate_b.reshape(1, 1),
                   fc1_W.T, fc1_b.reshape(1, D), fc2_W.T, fc2_b.reshape(1, D),
                   out_W.T, out_b.reshape(1, 1))
    return out.reshape(-1)
